# baseline (speedup 1.0000x reference)
import jax
import jax.numpy as jnp
from jax import lax
from jax.experimental import pallas as pl
from jax.experimental.pallas import tpu as pltpu
from jax.experimental.pallas import tpu_sc as plsc

B = 16384
L = 14
M = 512
P = 1024
NP1 = 16384
PAD = NP1 - 1
NW = 32
COLS = B // NW
EPW = L * COLS
CHUNK = 128
CPS = COLS // CHUNK

_f32 = jnp.float32
_i32 = jnp.int32


def _cm_body(t_ref, o_ref):
    o_ref[...] = t_ref[...]


def _make_cm(rows):
    return pl.pallas_call(
        _cm_body,
        grid=(NP1 // 128,),
        in_specs=[pl.BlockSpec((rows, 128), lambda j: (0, j))],
        out_specs=pl.BlockSpec((rows, 128), lambda j: (j, 0)),
        out_shape=jax.ShapeDtypeStruct((NP1 // 128 * rows, 128), _f32),
    )


_cm_meta = _make_cm(M)
_cm_pers = _make_cm(P)


def _sc_gather_body(paths_hbm, midx_hbm, pidx_hbm, bg_hbm, meta_hbm, pers_hbm,
                    bg_out, meta_out, pers_out,
                    paths_v, m_v, p_v, bg_v, im_v, ip_v, vm_v, vp_v, bga_v,
                    sem):
    wid = lax.axis_index("s") * 2 + lax.axis_index("c")
    base = wid * COLS

    pltpu.sync_copy(paths_hbm.at[:, pl.ds(base, COLS)], paths_v)
    pltpu.sync_copy(midx_hbm.at[pl.ds(base, COLS)], m_v)
    pltpu.sync_copy(pidx_hbm.at[pl.ds(base, COLS)], p_v)
    pltpu.sync_copy(bg_hbm, bg_v)

    def build(j, carry):
        l = j // 32
        c = j % 32
        n = paths_v[l, pl.ds(c * 16, 16)]
        m = m_v[pl.ds(c * 16, 16)]
        q = p_v[pl.ds(c * 16, 16)]
        npart = (n >> 7) << 7
        lane = n & 127
        im_v[l, pl.ds(c * 16, 16)] = ((npart * M + (m << 7)) + lane)
        ip_v[l, pl.ds(c * 16, 16)] = ((npart * P + (q << 7)) + lane)
        bga_v[l, pl.ds(c * 16, 16)] = plsc.load_gather(bg_v, [n])
        return carry

    lax.fori_loop(0, L * 32, build, 0)

    def fire(j, carry):
        l = j // CPS
        c = j % CPS
        pltpu.async_copy(meta_hbm.at[im_v.at[l, pl.ds(c * CHUNK, CHUNK)]],
                         vm_v.at[l, pl.ds(c * CHUNK, CHUNK)], sem)
        pltpu.async_copy(pers_hbm.at[ip_v.at[l, pl.ds(c * CHUNK, CHUNK)]],
                         vp_v.at[l, pl.ds(c * CHUNK, CHUNK)], sem)
        return carry

    lax.fori_loop(0, L * CPS, fire, 0)

    pltpu.sync_copy(bga_v, bg_out.at[:, pl.ds(base, COLS)])

    def drain(l, carry):
        pltpu.make_async_copy(meta_hbm.at[pl.ds(0, COLS)],
                              vm_v.at[l], sem).wait()
        pltpu.make_async_copy(pers_hbm.at[pl.ds(0, COLS)],
                              vp_v.at[l], sem).wait()
        return carry

    lax.fori_loop(0, L, drain, 0)
    pltpu.sync_copy(vm_v, meta_out.at[:, pl.ds(base, COLS)])
    pltpu.sync_copy(vp_v, pers_out.at[:, pl.ds(base, COLS)])


_sc_gather = pl.kernel(
    _sc_gather_body,
    out_type=[jax.ShapeDtypeStruct((L, B), _f32) for _ in range(3)],
    mesh=plsc.VectorSubcoreMesh(core_axis_name="c", subcore_axis_name="s"),
    compiler_params=pltpu.CompilerParams(needs_layout_passes=False),
    scratch_types=[
        pltpu.VMEM((L, COLS), _i32),
        pltpu.VMEM((COLS,), _i32),
        pltpu.VMEM((COLS,), _i32),
        pltpu.VMEM((NP1,), _f32),
        pltpu.VMEM((L, COLS), _i32),
        pltpu.VMEM((L, COLS), _i32),
        pltpu.VMEM((L, COLS), _f32),
        pltpu.VMEM((L, COLS), _f32),
        pltpu.VMEM((L, COLS), _f32),
        pltpu.SemaphoreType.DMA,
    ],
)


_TC_BLK = 2048


def _tc_finish_body(bg_ref, m_ref, p_ref, s_ref, q_ref, o_ref):
    x = s_ref[...] * (bg_ref[...] + m_ref[...] + p_ref[...])
    y = jnp.minimum(x, 0.0) - jnp.log(1.0 + jnp.exp(-jnp.abs(x)))
    z = y * (q_ref[...] != PAD).astype(_f32)
    o_ref[...] = jnp.sum(z, axis=0)


_tc_finish = pl.pallas_call(
    _tc_finish_body,
    grid=(B // _TC_BLK,),
    in_specs=[pl.BlockSpec((L, _TC_BLK), lambda i: (0, i)) for _ in range(5)],
    out_specs=pl.BlockSpec((_TC_BLK,), lambda i: (i,)),
    out_shape=jax.ShapeDtypeStruct((B,), _f32),
)


def kernel(r, m_idx, p_idx, node_paths, node_signs, eta_bg, eta_meta, eta_pers):
    del r
    paths_t = node_paths.T
    signs_t = node_signs.T
    meta_lin = _cm_meta(eta_meta).reshape(-1)
    pers_lin = _cm_pers(eta_pers).reshape(-1)
    bgv, mv, pv = _sc_gather(
        paths_t,
        m_idx.astype(_i32),
        p_idx.astype(_i32),
        eta_bg,
        meta_lin,
        pers_lin,
    )
    return _tc_finish(bgv, mv, pv, signs_t, paths_t)


# speedup vs baseline: 1.1631x; 1.1631x over previous
import jax
import jax.numpy as jnp
from jax import lax
from jax.experimental import pallas as pl
from jax.experimental.pallas import tpu as pltpu
from jax.experimental.pallas import tpu_sc as plsc

B = 16384
L = 14
M = 512
P = 1024
NP1 = 16384
PAD = NP1 - 1
NW = 32
COLS = B // NW
EPW = L * COLS
CHUNK = 128
CPS = COLS // CHUNK

_f32 = jnp.float32
_i32 = jnp.int32


def _cm_body(t_ref, o_ref):
    for c in range(NP1 // 128):
        o_ref[8 * c:8 * c + 8, :] = t_ref[:, 128 * c:128 * c + 128]


def _make_cm(rows):
    return pl.pallas_call(
        _cm_body,
        grid=(rows // 8,),
        in_specs=[pl.BlockSpec((8, NP1), lambda i: (i, 0))],
        out_specs=pl.BlockSpec((NP1 // 16, 128), lambda i: (i, 0)),
        out_shape=jax.ShapeDtypeStruct((rows * NP1 // 128, 128), _f32),
    )


_cm_meta = _make_cm(M)
_cm_pers = _make_cm(P)


def _sc_gather_body(paths_hbm, midx_hbm, pidx_hbm, bg_hbm, meta_hbm, pers_hbm,
                    bg_out, meta_out, pers_out,
                    paths_v, m_v, p_v, bg_v, im_v, ip_v, vm_v, vp_v, bga_v,
                    sem):
    wid = lax.axis_index("s") * 2 + lax.axis_index("c")
    base = wid * COLS

    pltpu.sync_copy(paths_hbm.at[:, pl.ds(base, COLS)], paths_v)
    pltpu.sync_copy(midx_hbm.at[pl.ds(base, COLS)], m_v)
    pltpu.sync_copy(pidx_hbm.at[pl.ds(base, COLS)], p_v)
    pltpu.sync_copy(bg_hbm, bg_v)

    def build(j, carry):
        l = j // 32
        c = j % 32
        n = paths_v[l, pl.ds(c * 16, 16)]
        m = m_v[pl.ds(c * 16, 16)]
        q = p_v[pl.ds(c * 16, 16)]
        npart = ((n >> 7) << 10) + (n & 127)
        im_v[l, pl.ds(c * 16, 16)] = ((m >> 3) << 17) + ((m & 7) << 7) + npart
        ip_v[l, pl.ds(c * 16, 16)] = ((q >> 3) << 17) + ((q & 7) << 7) + npart
        bga_v[l, pl.ds(c * 16, 16)] = plsc.load_gather(bg_v, [n])
        return carry

    lax.fori_loop(0, L * 32, build, 0)

    def fire(j, carry):
        l = j // CPS
        c = j % CPS
        pltpu.async_copy(meta_hbm.at[im_v.at[l, pl.ds(c * CHUNK, CHUNK)]],
                         vm_v.at[l, pl.ds(c * CHUNK, CHUNK)], sem)
        pltpu.async_copy(pers_hbm.at[ip_v.at[l, pl.ds(c * CHUNK, CHUNK)]],
                         vp_v.at[l, pl.ds(c * CHUNK, CHUNK)], sem)
        return carry

    lax.fori_loop(0, L * CPS, fire, 0)

    pltpu.sync_copy(bga_v, bg_out.at[:, pl.ds(base, COLS)])

    def drain(l, carry):
        pltpu.make_async_copy(meta_hbm.at[pl.ds(0, COLS)],
                              vm_v.at[l], sem).wait()
        pltpu.make_async_copy(pers_hbm.at[pl.ds(0, COLS)],
                              vp_v.at[l], sem).wait()
        return carry

    lax.fori_loop(0, L, drain, 0)
    pltpu.sync_copy(vm_v, meta_out.at[:, pl.ds(base, COLS)])
    pltpu.sync_copy(vp_v, pers_out.at[:, pl.ds(base, COLS)])


_sc_gather = pl.kernel(
    _sc_gather_body,
    out_type=[jax.ShapeDtypeStruct((L, B), _f32) for _ in range(3)],
    mesh=plsc.VectorSubcoreMesh(core_axis_name="c", subcore_axis_name="s"),
    compiler_params=pltpu.CompilerParams(needs_layout_passes=False),
    scratch_types=[
        pltpu.VMEM((L, COLS), _i32),
        pltpu.VMEM((COLS,), _i32),
        pltpu.VMEM((COLS,), _i32),
        pltpu.VMEM((NP1,), _f32),
        pltpu.VMEM((L, COLS), _i32),
        pltpu.VMEM((L, COLS), _i32),
        pltpu.VMEM((L, COLS), _f32),
        pltpu.VMEM((L, COLS), _f32),
        pltpu.VMEM((L, COLS), _f32),
        pltpu.SemaphoreType.DMA,
    ],
)


_TC_BLK = 2048


def _tc_finish_body(bg_ref, m_ref, p_ref, s_ref, q_ref, o_ref):
    x = s_ref[...] * (bg_ref[...] + m_ref[...] + p_ref[...])
    y = jnp.minimum(x, 0.0) - jnp.log(1.0 + jnp.exp(-jnp.abs(x)))
    z = y * (q_ref[...] != PAD).astype(_f32)
    o_ref[...] = jnp.sum(z, axis=0)


_tc_finish = pl.pallas_call(
    _tc_finish_body,
    grid=(B // _TC_BLK,),
    in_specs=[pl.BlockSpec((L, _TC_BLK), lambda i: (0, i)) for _ in range(5)],
    out_specs=pl.BlockSpec((_TC_BLK,), lambda i: (i,)),
    out_shape=jax.ShapeDtypeStruct((B,), _f32),
)


def kernel(r, m_idx, p_idx, node_paths, node_signs, eta_bg, eta_meta, eta_pers):
    del r
    paths_t = node_paths.T
    signs_t = node_signs.T
    meta_lin = _cm_meta(eta_meta).reshape(-1)
    pers_lin = _cm_pers(eta_pers).reshape(-1)
    bgv, mv, pv = _sc_gather(
        paths_t,
        m_idx.astype(_i32),
        p_idx.astype(_i32),
        eta_bg,
        meta_lin,
        pers_lin,
    )
    return _tc_finish(bgv, mv, pv, signs_t, paths_t)


# speedup vs baseline: 1.6961x; 1.4582x over previous
import jax
import jax.numpy as jnp
from jax import lax
from jax.experimental import pallas as pl
from jax.experimental.pallas import tpu as pltpu
from jax.experimental.pallas import tpu_sc as plsc

B = 16384
L = 14
M = 512
P = 1024
NP1 = 16384
PAD = NP1 - 1
NW = 32
COLS = B // NW
EPW = L * COLS
CHUNK = 128
CPS = COLS // CHUNK

_f32 = jnp.float32
_i32 = jnp.int32


def _cm_body(t_ref, o_ref):
    for c in range(NP1 // 128):
        o_ref[8 * c:8 * c + 8, :] = t_ref[:, 128 * c:128 * c + 128]


def _make_cm(rows):
    return pl.pallas_call(
        _cm_body,
        grid=(rows // 8,),
        in_specs=[pl.BlockSpec((8, NP1), lambda i: (i, 0))],
        out_specs=pl.BlockSpec((NP1 // 16, 128), lambda i: (i, 0)),
        out_shape=jax.ShapeDtypeStruct((rows * NP1 // 128, 128), _f32),
    )


_cm_meta = _make_cm(M)
_cm_pers = _make_cm(P)


def _sc_gather_body(paths_hbm, midx_hbm, pidx_hbm, bg_hbm, meta_hbm, pers_hbm,
                    bg_out, meta_out, pers_out,
                    paths_v, m_v, p_v, bg_v, im_v, ip_v, vm_v, vp_v, bga_v,
                    sem):
    wid = lax.axis_index("s") * 2 + lax.axis_index("c")
    base = wid * COLS

    pltpu.sync_copy(paths_hbm.at[:, pl.ds(base, COLS)], paths_v)
    pltpu.sync_copy(midx_hbm.at[pl.ds(base, COLS)], m_v)
    pltpu.sync_copy(pidx_hbm.at[pl.ds(base, COLS)], p_v)
    pltpu.sync_copy(bg_hbm, bg_v)

    def build(j, carry):
        l = j // 32
        c = j % 32
        n = paths_v[l, pl.ds(c * 16, 16)]
        m = m_v[pl.ds(c * 16, 16)]
        q = p_v[pl.ds(c * 16, 16)]
        im_v[l, pl.ds(c * 16, 16)] = m * NP1 + n
        ip_v[l, pl.ds(c * 16, 16)] = q * NP1 + n
        bga_v[l, pl.ds(c * 16, 16)] = plsc.load_gather(bg_v, [n])
        return carry

    lax.fori_loop(0, L * 32, build, 0)

    def fire(j, carry):
        l = j // CPS
        c = j % CPS
        pltpu.async_copy(meta_hbm.at[im_v.at[l, pl.ds(c * CHUNK, CHUNK)]],
                         vm_v.at[l, pl.ds(c * CHUNK, CHUNK)], sem)
        pltpu.async_copy(pers_hbm.at[ip_v.at[l, pl.ds(c * CHUNK, CHUNK)]],
                         vp_v.at[l, pl.ds(c * CHUNK, CHUNK)], sem)
        return carry

    lax.fori_loop(0, L * CPS, fire, 0)

    pltpu.sync_copy(bga_v, bg_out.at[:, pl.ds(base, COLS)])

    def drain(l, carry):
        pltpu.make_async_copy(meta_hbm.at[pl.ds(0, COLS)],
                              vm_v.at[l], sem).wait()
        pltpu.make_async_copy(pers_hbm.at[pl.ds(0, COLS)],
                              vp_v.at[l], sem).wait()
        return carry

    lax.fori_loop(0, L, drain, 0)
    pltpu.sync_copy(vm_v, meta_out.at[:, pl.ds(base, COLS)])
    pltpu.sync_copy(vp_v, pers_out.at[:, pl.ds(base, COLS)])


_sc_gather = pl.kernel(
    _sc_gather_body,
    out_type=[jax.ShapeDtypeStruct((L, B), _f32) for _ in range(3)],
    mesh=plsc.VectorSubcoreMesh(core_axis_name="c", subcore_axis_name="s"),
    compiler_params=pltpu.CompilerParams(needs_layout_passes=False),
    scratch_types=[
        pltpu.VMEM((L, COLS), _i32),
        pltpu.VMEM((COLS,), _i32),
        pltpu.VMEM((COLS,), _i32),
        pltpu.VMEM((NP1,), _f32),
        pltpu.VMEM((L, COLS), _i32),
        pltpu.VMEM((L, COLS), _i32),
        pltpu.VMEM((L, COLS), _f32),
        pltpu.VMEM((L, COLS), _f32),
        pltpu.VMEM((L, COLS), _f32),
        pltpu.SemaphoreType.DMA,
    ],
)


_TC_BLK = 2048


def _tc_finish_body(bg_ref, m_ref, p_ref, s_ref, q_ref, o_ref):
    x = s_ref[...] * (bg_ref[...] + m_ref[...] + p_ref[...])
    y = jnp.minimum(x, 0.0) - jnp.log(1.0 + jnp.exp(-jnp.abs(x)))
    z = y * (q_ref[...] != PAD).astype(_f32)
    o_ref[...] = jnp.sum(z, axis=0)


_tc_finish = pl.pallas_call(
    _tc_finish_body,
    grid=(B // _TC_BLK,),
    in_specs=[pl.BlockSpec((L, _TC_BLK), lambda i: (0, i)) for _ in range(5)],
    out_specs=pl.BlockSpec((_TC_BLK,), lambda i: (i,)),
    out_shape=jax.ShapeDtypeStruct((B,), _f32),
)


def kernel(r, m_idx, p_idx, node_paths, node_signs, eta_bg, eta_meta, eta_pers):
    del r
    paths_t = node_paths.T
    signs_t = node_signs.T
    meta_lin = eta_meta.reshape(-1)
    pers_lin = eta_pers.reshape(-1)
    bgv, mv, pv = _sc_gather(
        paths_t,
        m_idx.astype(_i32),
        p_idx.astype(_i32),
        eta_bg,
        meta_lin,
        pers_lin,
    )
    return _tc_finish(bgv, mv, pv, signs_t, paths_t)


# speedup vs baseline: 1.7515x; 1.0327x over previous
import jax
import jax.numpy as jnp
from jax import lax
from jax.experimental import pallas as pl
from jax.experimental.pallas import tpu as pltpu
from jax.experimental.pallas import tpu_sc as plsc

B = 16384
L = 14
M = 512
P = 1024
NP1 = 16384
PAD = NP1 - 1
NW = 32
COLS = B // NW
EPW = L * COLS
CHUNK = 128
CPS = COLS // CHUNK

_f32 = jnp.float32
_i32 = jnp.int32


def _cm_body(t_ref, o_ref):
    for c in range(NP1 // 128):
        o_ref[8 * c:8 * c + 8, :] = t_ref[:, 128 * c:128 * c + 128]


def _make_cm(rows):
    return pl.pallas_call(
        _cm_body,
        grid=(rows // 8,),
        in_specs=[pl.BlockSpec((8, NP1), lambda i: (i, 0))],
        out_specs=pl.BlockSpec((NP1 // 16, 128), lambda i: (i, 0)),
        out_shape=jax.ShapeDtypeStruct((rows * NP1 // 128, 128), _f32),
    )


_cm_meta = _make_cm(M)


def _sc_gather_body(paths_hbm, midx_hbm, pidx_hbm, bg_hbm, meta_hbm, pers_hbm,
                    bg_out, meta_out, pers_out,
                    paths_v, m_v, p_v, bg_v, im_v, ip_v, vm_v, vp_v, bga_v,
                    sem):
    wid = lax.axis_index("s") * 2 + lax.axis_index("c")
    base = wid * COLS

    pltpu.sync_copy(paths_hbm.at[:, pl.ds(base, COLS)], paths_v)
    pltpu.sync_copy(midx_hbm.at[pl.ds(base, COLS)], m_v)
    pltpu.sync_copy(pidx_hbm.at[pl.ds(base, COLS)], p_v)
    pltpu.sync_copy(bg_hbm, bg_v)

    def build(j, carry):
        l = j // 32
        c = j % 32
        n = paths_v[l, pl.ds(c * 16, 16)]
        m = m_v[pl.ds(c * 16, 16)]
        q = p_v[pl.ds(c * 16, 16)]
        im_v[l, pl.ds(c * 16, 16)] = (((m >> 3) << 17) + ((m & 7) << 7)
                                      + ((n >> 7) << 10) + (n & 127))
        ip_v[l, pl.ds(c * 16, 16)] = q * NP1 + n
        bga_v[l, pl.ds(c * 16, 16)] = plsc.load_gather(bg_v, [n])
        return carry

    lax.fori_loop(0, L * 32, build, 0)

    def fire(j, carry):
        l = j // CPS
        c = j % CPS
        pltpu.async_copy(meta_hbm.at[im_v.at[l, pl.ds(c * CHUNK, CHUNK)]],
                         vm_v.at[l, pl.ds(c * CHUNK, CHUNK)], sem)
        pltpu.async_copy(pers_hbm.at[ip_v.at[l, pl.ds(c * CHUNK, CHUNK)]],
                         vp_v.at[l, pl.ds(c * CHUNK, CHUNK)], sem)
        return carry

    lax.fori_loop(0, L * CPS, fire, 0)

    pltpu.sync_copy(bga_v, bg_out.at[:, pl.ds(base, COLS)])

    def drain(l, carry):
        pltpu.make_async_copy(meta_hbm.at[pl.ds(0, COLS)],
                              vm_v.at[l], sem).wait()
        pltpu.make_async_copy(pers_hbm.at[pl.ds(0, COLS)],
                              vp_v.at[l], sem).wait()
        return carry

    lax.fori_loop(0, L, drain, 0)
    pltpu.sync_copy(vm_v, meta_out.at[:, pl.ds(base, COLS)])
    pltpu.sync_copy(vp_v, pers_out.at[:, pl.ds(base, COLS)])


_sc_gather = pl.kernel(
    _sc_gather_body,
    out_type=[jax.ShapeDtypeStruct((L, B), _f32) for _ in range(3)],
    mesh=plsc.VectorSubcoreMesh(core_axis_name="c", subcore_axis_name="s"),
    compiler_params=pltpu.CompilerParams(needs_layout_passes=False),
    scratch_types=[
        pltpu.VMEM((L, COLS), _i32),
        pltpu.VMEM((COLS,), _i32),
        pltpu.VMEM((COLS,), _i32),
        pltpu.VMEM((NP1,), _f32),
        pltpu.VMEM((L, COLS), _i32),
        pltpu.VMEM((L, COLS), _i32),
        pltpu.VMEM((L, COLS), _f32),
        pltpu.VMEM((L, COLS), _f32),
        pltpu.VMEM((L, COLS), _f32),
        pltpu.SemaphoreType.DMA,
    ],
)


_TC_BLK = 2048


def _tc_finish_body(bg_ref, m_ref, p_ref, s_ref, q_ref, o_ref):
    x = s_ref[...] * (bg_ref[...] + m_ref[...] + p_ref[...])
    y = jnp.minimum(x, 0.0) - jnp.log(1.0 + jnp.exp(-jnp.abs(x)))
    z = y * (q_ref[...] != PAD).astype(_f32)
    o_ref[...] = jnp.sum(z, axis=0)


_tc_finish = pl.pallas_call(
    _tc_finish_body,
    grid=(B // _TC_BLK,),
    in_specs=[pl.BlockSpec((L, _TC_BLK), lambda i: (0, i)) for _ in range(5)],
    out_specs=pl.BlockSpec((_TC_BLK,), lambda i: (i,)),
    out_shape=jax.ShapeDtypeStruct((B,), _f32),
)


def kernel(r, m_idx, p_idx, node_paths, node_signs, eta_bg, eta_meta, eta_pers):
    del r
    paths_t = node_paths.T
    signs_t = node_signs.T
    meta_lin = _cm_meta(eta_meta).reshape(-1)
    pers_lin = eta_pers.reshape(-1)
    bgv, mv, pv = _sc_gather(
        paths_t,
        m_idx.astype(_i32),
        p_idx.astype(_i32),
        eta_bg,
        meta_lin,
        pers_lin,
    )
    return _tc_finish(bgv, mv, pv, signs_t, paths_t)


# speedup vs baseline: 1.7532x; 1.0010x over previous
import jax
import jax.numpy as jnp
from jax import lax
from jax.experimental import pallas as pl
from jax.experimental.pallas import tpu as pltpu
from jax.experimental.pallas import tpu_sc as plsc

B = 16384
L = 14
M = 512
P = 1024
NP1 = 16384
PAD = NP1 - 1
NW = 32
COLS = B // NW
EPW = L * COLS
CHUNK = 128
CPS = COLS // CHUNK

_f32 = jnp.float32
_i32 = jnp.int32


def _sc_gather_body(paths_hbm, midx_hbm, pidx_hbm, bg_hbm, meta_hbm, pers_hbm,
                    bg_out, meta_out, pers_out,
                    paths_v, m_v, p_v, bg_v, im_v, ip_v, vm_v, vp_v, bga_v,
                    sem):
    wid = lax.axis_index("s") * 2 + lax.axis_index("c")
    base = wid * COLS

    pltpu.sync_copy(paths_hbm.at[:, pl.ds(base, COLS)], paths_v)
    pltpu.sync_copy(midx_hbm.at[pl.ds(base, COLS)], m_v)
    pltpu.sync_copy(pidx_hbm.at[pl.ds(base, COLS)], p_v)
    pltpu.sync_copy(bg_hbm, bg_v)

    def build_fire(j, carry):
        l = j // CPS
        c = j % CPS
        for k in range(CHUNK // 16):
            o = c * CHUNK + k * 16
            n = paths_v[l, pl.ds(o, 16)]
            m = m_v[pl.ds(o, 16)]
            q = p_v[pl.ds(o, 16)]
            im_v[l, pl.ds(o, 16)] = m * NP1 + n
            ip_v[l, pl.ds(o, 16)] = q * NP1 + n
            bga_v[l, pl.ds(o, 16)] = plsc.load_gather(bg_v, [n])
        pltpu.async_copy(meta_hbm.at[im_v.at[l, pl.ds(c * CHUNK, CHUNK)]],
                         vm_v.at[l, pl.ds(c * CHUNK, CHUNK)], sem)
        pltpu.async_copy(pers_hbm.at[ip_v.at[l, pl.ds(c * CHUNK, CHUNK)]],
                         vp_v.at[l, pl.ds(c * CHUNK, CHUNK)], sem)
        return carry

    lax.fori_loop(0, L * CPS, build_fire, 0)

    pltpu.sync_copy(bga_v, bg_out.at[:, pl.ds(base, COLS)])

    def drain(l, carry):
        pltpu.make_async_copy(meta_hbm.at[pl.ds(0, COLS)],
                              vm_v.at[l], sem).wait()
        pltpu.make_async_copy(pers_hbm.at[pl.ds(0, COLS)],
                              vp_v.at[l], sem).wait()
        return carry

    lax.fori_loop(0, L, drain, 0)
    pltpu.sync_copy(vm_v, meta_out.at[:, pl.ds(base, COLS)])
    pltpu.sync_copy(vp_v, pers_out.at[:, pl.ds(base, COLS)])


_sc_gather = pl.kernel(
    _sc_gather_body,
    out_type=[jax.ShapeDtypeStruct((L, B), _f32) for _ in range(3)],
    mesh=plsc.VectorSubcoreMesh(core_axis_name="c", subcore_axis_name="s"),
    compiler_params=pltpu.CompilerParams(needs_layout_passes=False),
    scratch_types=[
        pltpu.VMEM((L, COLS), _i32),
        pltpu.VMEM((COLS,), _i32),
        pltpu.VMEM((COLS,), _i32),
        pltpu.VMEM((NP1,), _f32),
        pltpu.VMEM((L, COLS), _i32),
        pltpu.VMEM((L, COLS), _i32),
        pltpu.VMEM((L, COLS), _f32),
        pltpu.VMEM((L, COLS), _f32),
        pltpu.VMEM((L, COLS), _f32),
        pltpu.SemaphoreType.DMA,
    ],
)


_TC_BLK = 2048


def _tc_finish_body(bg_ref, m_ref, p_ref, s_ref, q_ref, o_ref):
    x = s_ref[...] * (bg_ref[...] + m_ref[...] + p_ref[...])
    y = jnp.minimum(x, 0.0) - jnp.log(1.0 + jnp.exp(-jnp.abs(x)))
    z = y * (q_ref[...] != PAD).astype(_f32)
    o_ref[...] = jnp.sum(z, axis=0)


_tc_finish = pl.pallas_call(
    _tc_finish_body,
    grid=(B // _TC_BLK,),
    in_specs=[pl.BlockSpec((L, _TC_BLK), lambda i: (0, i)) for _ in range(5)],
    out_specs=pl.BlockSpec((_TC_BLK,), lambda i: (i,)),
    out_shape=jax.ShapeDtypeStruct((B,), _f32),
)


def kernel(r, m_idx, p_idx, node_paths, node_signs, eta_bg, eta_meta, eta_pers):
    del r
    paths_t = node_paths.T
    signs_t = node_signs.T
    meta_lin = eta_meta.reshape(-1)
    pers_lin = eta_pers.reshape(-1)
    bgv, mv, pv = _sc_gather(
        paths_t,
        m_idx.astype(_i32),
        p_idx.astype(_i32),
        eta_bg,
        meta_lin,
        pers_lin,
    )
    return _tc_finish(bgv, mv, pv, signs_t, paths_t)


# speedup vs baseline: 1.7685x; 1.0087x over previous
import jax
import jax.numpy as jnp
from jax import lax
from jax.experimental import pallas as pl
from jax.experimental.pallas import tpu as pltpu
from jax.experimental.pallas import tpu_sc as plsc

B = 16384
L = 14
M = 512
P = 1024
NP1 = 16384
PAD = NP1 - 1
NW = 32
COLS = B // NW
EPW = L * COLS
CHUNK = 128
CPS = COLS // CHUNK

_f32 = jnp.float32
_i32 = jnp.int32


def _sc_gather_body(paths_hbm, midx_hbm, pidx_hbm, bg_hbm, meta_hbm, pers_hbm,
                    bg_out, meta_out, pers_out,
                    paths_v, m_v, p_v, bg_v, im_v, ip_v, vm_v, vp_v, bga_v,
                    sem):
    wid = lax.axis_index("s") * 2 + lax.axis_index("c")
    base = wid * COLS

    cps = [
        pltpu.async_copy(paths_hbm.at[:, pl.ds(base, COLS)], paths_v, sem),
        pltpu.async_copy(midx_hbm.at[pl.ds(base, COLS)], m_v, sem),
        pltpu.async_copy(pidx_hbm.at[pl.ds(base, COLS)], p_v, sem),
        pltpu.async_copy(bg_hbm, bg_v, sem),
    ]
    for cp in cps:
        cp.wait()

    def build_fire(j, carry):
        l = j // CPS
        c = j % CPS
        for k in range(CHUNK // 16):
            o = c * CHUNK + k * 16
            n = paths_v[l, pl.ds(o, 16)]
            m = m_v[pl.ds(o, 16)]
            q = p_v[pl.ds(o, 16)]
            im_v[l, pl.ds(o, 16)] = m * NP1 + n
            ip_v[l, pl.ds(o, 16)] = q * NP1 + n
            bga_v[l, pl.ds(o, 16)] = plsc.load_gather(bg_v, [n])
        pltpu.async_copy(meta_hbm.at[im_v.at[l, pl.ds(c * CHUNK, CHUNK)]],
                         vm_v.at[l, pl.ds(c * CHUNK, CHUNK)], sem)
        pltpu.async_copy(pers_hbm.at[ip_v.at[l, pl.ds(c * CHUNK, CHUNK)]],
                         vp_v.at[l, pl.ds(c * CHUNK, CHUNK)], sem)
        return carry

    lax.fori_loop(0, L * CPS, build_fire, 0)

    pltpu.sync_copy(bga_v, bg_out.at[:, pl.ds(base, COLS)])

    def drain(l, carry):
        pltpu.make_async_copy(meta_hbm.at[pl.ds(0, COLS)],
                              vm_v.at[l], sem).wait()
        pltpu.make_async_copy(pers_hbm.at[pl.ds(0, COLS)],
                              vp_v.at[l], sem).wait()
        return carry

    lax.fori_loop(0, L, drain, 0)
    pltpu.sync_copy(vm_v, meta_out.at[:, pl.ds(base, COLS)])
    pltpu.sync_copy(vp_v, pers_out.at[:, pl.ds(base, COLS)])


_sc_gather = pl.kernel(
    _sc_gather_body,
    out_type=[jax.ShapeDtypeStruct((L, B), _f32) for _ in range(3)],
    mesh=plsc.VectorSubcoreMesh(core_axis_name="c", subcore_axis_name="s"),
    compiler_params=pltpu.CompilerParams(needs_layout_passes=False),
    scratch_types=[
        pltpu.VMEM((L, COLS), _i32),
        pltpu.VMEM((COLS,), _i32),
        pltpu.VMEM((COLS,), _i32),
        pltpu.VMEM((NP1,), _f32),
        pltpu.VMEM((L, COLS), _i32),
        pltpu.VMEM((L, COLS), _i32),
        pltpu.VMEM((L, COLS), _f32),
        pltpu.VMEM((L, COLS), _f32),
        pltpu.VMEM((L, COLS), _f32),
        pltpu.SemaphoreType.DMA,
    ],
)


_TC_BLK = 2048


def _tc_finish_body(bg_ref, m_ref, p_ref, s_ref, q_ref, o_ref):
    x = s_ref[...] * (bg_ref[...] + m_ref[...] + p_ref[...])
    y = jnp.minimum(x, 0.0) - jnp.log(1.0 + jnp.exp(-jnp.abs(x)))
    z = y * (q_ref[...] != PAD).astype(_f32)
    o_ref[...] = jnp.sum(z, axis=0)


_tc_finish = pl.pallas_call(
    _tc_finish_body,
    grid=(B // _TC_BLK,),
    in_specs=[pl.BlockSpec((L, _TC_BLK), lambda i: (0, i)) for _ in range(5)],
    out_specs=pl.BlockSpec((_TC_BLK,), lambda i: (i,)),
    out_shape=jax.ShapeDtypeStruct((B,), _f32),
)


def kernel(r, m_idx, p_idx, node_paths, node_signs, eta_bg, eta_meta, eta_pers):
    del r
    paths_t = node_paths.T
    signs_t = node_signs.T
    meta_lin = eta_meta.reshape(-1)
    pers_lin = eta_pers.reshape(-1)
    bgv, mv, pv = _sc_gather(
        paths_t,
        m_idx.astype(_i32),
        p_idx.astype(_i32),
        eta_bg,
        meta_lin,
        pers_lin,
    )
    return _tc_finish(bgv, mv, pv, signs_t, paths_t)


# speedup vs baseline: 1.7693x; 1.0004x over previous
import jax
import jax.numpy as jnp
from jax import lax
from jax.experimental import pallas as pl
from jax.experimental.pallas import tpu as pltpu
from jax.experimental.pallas import tpu_sc as plsc

B = 16384
L = 14
M = 512
P = 1024
NP1 = 16384
PAD = NP1 - 1
NW = 32
COLS = B // NW
EPW = L * COLS
CHUNK = 128
CPS = COLS // CHUNK

_f32 = jnp.float32
_i32 = jnp.int32


def _sc_gather_body(paths_hbm, midx_hbm, pidx_hbm, bg_hbm, meta_hbm, pers_hbm,
                    bg_out, meta_out, pers_out,
                    paths_v, m_v, p_v, bg_v, im_v, ip_v, vm_v, vp_v, bga_v,
                    sem):
    wid = lax.axis_index("s") * 2 + lax.axis_index("c")
    base = wid * COLS

    cps = [
        pltpu.async_copy(paths_hbm.at[:, pl.ds(base, COLS)], paths_v, sem),
        pltpu.async_copy(midx_hbm.at[pl.ds(base, COLS)], m_v, sem),
        pltpu.async_copy(pidx_hbm.at[pl.ds(base, COLS)], p_v, sem),
        pltpu.async_copy(bg_hbm, bg_v, sem),
    ]
    for cp in cps:
        cp.wait()

    def build_fire(j, carry):
        l = j // CPS
        c = j % CPS
        for k in range(CHUNK // 16):
            o = c * CHUNK + k * 16
            n = paths_v[l, pl.ds(o, 16)]
            m = m_v[pl.ds(o, 16)]
            q = p_v[pl.ds(o, 16)]
            im_v[l, pl.ds(o, 16)] = m * NP1 + n
            ip_v[l, pl.ds(o, 16)] = q * NP1 + n
        pltpu.async_copy(meta_hbm.at[im_v.at[l, pl.ds(c * CHUNK, CHUNK)]],
                         vm_v.at[l, pl.ds(c * CHUNK, CHUNK)], sem)
        pltpu.async_copy(pers_hbm.at[ip_v.at[l, pl.ds(c * CHUNK, CHUNK)]],
                         vp_v.at[l, pl.ds(c * CHUNK, CHUNK)], sem)
        return carry

    lax.fori_loop(0, L * CPS, build_fire, 0)

    def bgather(j, carry):
        l = j // 32
        o = (j % 32) * 16
        n = paths_v[l, pl.ds(o, 16)]
        bga_v[l, pl.ds(o, 16)] = plsc.load_gather(bg_v, [n])
        return carry

    lax.fori_loop(0, L * 32, bgather, 0)
    pltpu.sync_copy(bga_v, bg_out.at[:, pl.ds(base, COLS)])

    def drain(l, carry):
        pltpu.make_async_copy(meta_hbm.at[pl.ds(0, COLS)],
                              vm_v.at[l], sem).wait()
        pltpu.make_async_copy(pers_hbm.at[pl.ds(0, COLS)],
                              vp_v.at[l], sem).wait()
        return carry

    lax.fori_loop(0, L, drain, 0)
    pltpu.sync_copy(vm_v, meta_out.at[:, pl.ds(base, COLS)])
    pltpu.sync_copy(vp_v, pers_out.at[:, pl.ds(base, COLS)])


_sc_gather = pl.kernel(
    _sc_gather_body,
    out_type=[jax.ShapeDtypeStruct((L, B), _f32) for _ in range(3)],
    mesh=plsc.VectorSubcoreMesh(core_axis_name="c", subcore_axis_name="s"),
    compiler_params=pltpu.CompilerParams(needs_layout_passes=False),
    scratch_types=[
        pltpu.VMEM((L, COLS), _i32),
        pltpu.VMEM((COLS,), _i32),
        pltpu.VMEM((COLS,), _i32),
        pltpu.VMEM((NP1,), _f32),
        pltpu.VMEM((L, COLS), _i32),
        pltpu.VMEM((L, COLS), _i32),
        pltpu.VMEM((L, COLS), _f32),
        pltpu.VMEM((L, COLS), _f32),
        pltpu.VMEM((L, COLS), _f32),
        pltpu.SemaphoreType.DMA,
    ],
)


_TC_BLK = 2048


def _tc_finish_body(bg_ref, m_ref, p_ref, s_ref, q_ref, o_ref):
    x = s_ref[...] * (bg_ref[...] + m_ref[...] + p_ref[...])
    y = jnp.minimum(x, 0.0) - jnp.log(1.0 + jnp.exp(-jnp.abs(x)))
    z = y * (q_ref[...] != PAD).astype(_f32)
    o_ref[...] = jnp.sum(z, axis=0)


_tc_finish = pl.pallas_call(
    _tc_finish_body,
    grid=(B // _TC_BLK,),
    in_specs=[pl.BlockSpec((L, _TC_BLK), lambda i: (0, i)) for _ in range(5)],
    out_specs=pl.BlockSpec((_TC_BLK,), lambda i: (i,)),
    out_shape=jax.ShapeDtypeStruct((B,), _f32),
)


def kernel(r, m_idx, p_idx, node_paths, node_signs, eta_bg, eta_meta, eta_pers):
    del r
    paths_t = node_paths.T
    signs_t = node_signs.T
    meta_lin = eta_meta.reshape(-1)
    pers_lin = eta_pers.reshape(-1)
    bgv, mv, pv = _sc_gather(
        paths_t,
        m_idx.astype(_i32),
        p_idx.astype(_i32),
        eta_bg,
        meta_lin,
        pers_lin,
    )
    return _tc_finish(bgv, mv, pv, signs_t, paths_t)
